# baseline (speedup 1.0000x reference)
import functools

import jax
import jax.numpy as jnp
from jax import lax
from jax.experimental import pallas as pl
from jax.experimental.pallas import tpu as pltpu
from jax.experimental.pallas import tpu_sc as plsc

_RB = 2


def _vq_tc_body(nb, d, z_ref, w_ref, idx_ref, loss_ref):
    pid = pl.program_id(0)
    rb, s, _ = z_ref.shape
    bt = rb * s
    zb = z_ref[...].reshape(bt, d)
    w = w_ref[...]
    k = w.shape[0]
    wsq = jnp.sum(w * w, axis=1)
    zsq = jnp.sum(zb * zb, axis=1)
    mm = lax.dot_general(zb, w, (((1,), (1,)), ((), ())),
                         preferred_element_type=jnp.float32)
    dist = (zsq[:, None] + wsq[None, :]) - 2.0 * mm
    m = jnp.min(dist, axis=1)
    ii = lax.broadcasted_iota(jnp.int32, dist.shape, 1)
    idx = jnp.min(jnp.where(dist == m[:, None], ii, k), axis=1)
    idx_ref[...] = idx

    @pl.when(pid == 0)
    def _init():
        loss_ref[...] = jnp.zeros((1, 1), jnp.float32)

    loss_ref[...] += jnp.sum(m).reshape(1, 1)

    @pl.when(pid == pl.num_programs(0) - 1)
    def _finish():
        loss_ref[...] *= 1.25 / (nb * s * d)


def _sc_gather(w, idx, b, s):
    k, d = w.shape
    ch = 128
    n_ch = s // ch
    info = plsc.get_sparse_core_info()
    mesh = plsc.VectorSubcoreMesh(core_axis_name="c", subcore_axis_name="s")

    @functools.partial(
        pl.kernel, mesh=mesh,
        out_type=jax.ShapeDtypeStruct((b, s, d), jnp.float32),
        compiler_params=pltpu.CompilerParams(use_tc_tiling_on_sc=False),
        scratch_types=[
            pltpu.VMEM((s,), jnp.int32),
            pltpu.VMEM((s, d), jnp.float32),
            pltpu.SemaphoreType.DMA,
        ],
    )
    def gk(w_hbm, idx_hbm, out_hbm, idx_v, rows_v, sem):
        wid = lax.axis_index("s") * info.num_cores + lax.axis_index("c")
        pltpu.sync_copy(idx_hbm.at[pl.ds(wid * s, s)], idx_v)
        copies = [
            pltpu.async_copy(w_hbm.at[idx_v.at[pl.ds(c * ch, ch)]],
                             rows_v.at[pl.ds(c * ch, ch)], sem)
            for c in range(n_ch)
        ]
        for cp in copies:
            cp.wait()
        pltpu.sync_copy(rows_v, out_hbm.at[wid])

    return gk(w, idx)


def kernel(z, W):
    b, s, d = z.shape
    k = W.shape[0]
    n_blk = b // _RB

    idx, loss_acc = pl.pallas_call(
        functools.partial(_vq_tc_body, b, d),
        grid=(n_blk,),
        in_specs=[
            pl.BlockSpec((_RB, s, d), lambda i: (i, 0, 0)),
            pl.BlockSpec((k, d), lambda i: (0, 0)),
        ],
        out_specs=[
            pl.BlockSpec((_RB * s,), lambda i: (i,)),
            pl.BlockSpec((1, 1), lambda i: (0, 0)),
        ],
        out_shape=[
            jax.ShapeDtypeStruct((b * s,), jnp.int32),
            jax.ShapeDtypeStruct((1, 1), jnp.float32),
        ],
    )(z, W)

    q = _sc_gather(W, idx, b, s)

    return q, loss_acc[0, 0], idx.reshape(b, s)


# speedup vs baseline: 1.4638x; 1.4638x over previous
import functools

import jax
import jax.numpy as jnp
from jax import lax
from jax.experimental import pallas as pl
from jax.experimental.pallas import tpu as pltpu
from jax.experimental.pallas import tpu_sc as plsc

_CH = 128


def _vq_tc_body(nb, d, zt_ref, wt_ref, idx_ref, loss_ref):
    pid = pl.program_id(0)
    zt = zt_ref[...].reshape(d, zt_ref.shape[-1])
    wt = wt_ref[...]
    s = zt.shape[1]
    k = wt.shape[1]
    zsq = jnp.sum(zt * zt, axis=0)
    wsq = jnp.sum(wt * wt, axis=0)
    wsq_col = wsq.reshape(k, 1)
    mm = lax.dot_general(wt, zt, (((0,), (0,)), ((), ())),
                         preferred_element_type=jnp.float32)
    dist = (zsq[None, :] + wsq_col) - 2.0 * mm
    val = dist[0:_CH, :]
    chk = jnp.zeros((_CH, s), jnp.float32)
    for j in range(1, k // _CH):
        dj = dist[j * _CH:(j + 1) * _CH, :]
        cond = dj < val
        val = jnp.minimum(val, dj)
        chk = jnp.where(cond, jnp.float32(j), chk)
    m = jnp.min(val, axis=0)
    row_f = lax.broadcasted_iota(jnp.int32, (_CH, s), 0).astype(jnp.float32)
    g = chk * jnp.float32(_CH) + row_f
    idxf = jnp.min(jnp.where(val == m[None, :], g, jnp.float32(k)), axis=0)
    idx_ref[...] = idxf.astype(jnp.int32)

    @pl.when(pid == 0)
    def _init():
        loss_ref[...] = jnp.zeros((1, 1), jnp.float32)

    loss_ref[...] += jnp.sum(m).reshape(1, 1)

    @pl.when(pid == pl.num_programs(0) - 1)
    def _finish():
        loss_ref[...] *= 1.25 / (nb * s * d)


def _sc_gather_t(wt, idx, b, s):
    d, k = wt.shape
    hd = d // 2
    info = plsc.get_sparse_core_info()
    nl = info.num_lanes
    mesh = plsc.VectorSubcoreMesh(core_axis_name="c", subcore_axis_name="s")

    @functools.partial(
        pl.kernel, mesh=mesh,
        out_type=jax.ShapeDtypeStruct((b, d, s), jnp.float32),
        compiler_params=pltpu.CompilerParams(use_tc_tiling_on_sc=False,
                                             needs_layout_passes=False),
        scratch_types=[
            pltpu.VMEM((s,), jnp.int32),
            pltpu.VMEM((d, k), jnp.float32),
            pltpu.VMEM((hd, s), jnp.float32),
        ],
    )
    def gk(wt_hbm, idx_hbm, out_hbm, idx_v, wt_v, out_v):
        wid = lax.axis_index("s") * info.num_cores + lax.axis_index("c")
        pltpu.sync_copy(wt_hbm, wt_v)
        pltpu.sync_copy(idx_hbm.at[pl.ds(wid * s, s)], idx_v)
        for h in range(2):
            for dd in range(hd):
                row = jnp.full((nl,), h * hd + dd, jnp.int32)

                @plsc.parallel_loop(0, s // nl, unroll=4)
                def _gather_row(gg, dd=dd, row=row):
                    toks = idx_v[pl.ds(gg * nl, nl)]
                    vals = plsc.load_gather(wt_v, [row, toks])
                    out_v[dd, pl.ds(gg * nl, nl)] = vals
            pltpu.sync_copy(out_v, out_hbm.at[wid, pl.ds(h * hd, hd)])

    return gk(wt, idx)


def kernel(z, W):
    b, s, d = z.shape
    k = W.shape[0]
    zt = jnp.transpose(z, (0, 2, 1))
    wt = jnp.transpose(W, (1, 0))

    idx, loss_acc = pl.pallas_call(
        functools.partial(_vq_tc_body, b, d),
        grid=(b,),
        in_specs=[
            pl.BlockSpec((1, d, s), lambda i: (i, 0, 0)),
            pl.BlockSpec((d, k), lambda i: (0, 0)),
        ],
        out_specs=[
            pl.BlockSpec((s,), lambda i: (i,)),
            pl.BlockSpec((1, 1), lambda i: (0, 0)),
        ],
        out_shape=[
            jax.ShapeDtypeStruct((b * s,), jnp.int32),
            jax.ShapeDtypeStruct((1, 1), jnp.float32),
        ],
    )(zt, wt)

    qt = _sc_gather_t(wt, idx, b, s)

    return (jnp.transpose(qt, (0, 2, 1)), loss_acc[0, 0],
            idx.reshape(b, s))


# speedup vs baseline: 1.5876x; 1.0846x over previous
import functools

import jax
import jax.numpy as jnp
from jax import lax
from jax.experimental import pallas as pl
from jax.experimental.pallas import tpu as pltpu
from jax.experimental.pallas import tpu_sc as plsc

_CH = 128


def _vq_tc_body(nb, d, zt_ref, wt_ref, idx_ref, loss_ref):
    pid = pl.program_id(0)
    zt = zt_ref[...].reshape(d, zt_ref.shape[-1])
    wt = wt_ref[...]
    s = zt.shape[1]
    k = wt.shape[1]
    zsq = jnp.sum(zt * zt, axis=0)
    wsq = jnp.sum(wt * wt, axis=0)
    wsq_col = wsq.reshape(k, 1)
    mm = lax.dot_general(wt, zt, (((0,), (0,)), ((), ())),
                         preferred_element_type=jnp.float32)
    dist = (zsq[None, :] + wsq_col) - 2.0 * mm
    val = dist[0:_CH, :]
    chk = jnp.zeros((_CH, s), jnp.float32)
    for j in range(1, k // _CH):
        dj = dist[j * _CH:(j + 1) * _CH, :]
        cond = dj < val
        val = jnp.minimum(val, dj)
        chk = jnp.where(cond, jnp.float32(j), chk)
    m = jnp.min(val, axis=0)
    row_f = lax.broadcasted_iota(jnp.int32, (_CH, s), 0).astype(jnp.float32)
    g = chk * jnp.float32(_CH) + row_f
    idxf = jnp.min(jnp.where(val == m[None, :], g, jnp.float32(k)), axis=0)
    idx_ref[...] = idxf.astype(jnp.int32)

    @pl.when(pid == 0)
    def _init():
        loss_ref[...] = jnp.zeros((1, 1), jnp.float32)

    loss_ref[...] += jnp.sum(m).reshape(1, 1)

    @pl.when(pid == pl.num_programs(0) - 1)
    def _finish():
        loss_ref[...] *= 1.25 / (nb * s * d)


def _sc_gather_t(wt, idx, b, s):
    d, k = wt.shape
    hd = d // 2
    info = plsc.get_sparse_core_info()
    nl = info.num_lanes
    mesh = plsc.VectorSubcoreMesh(core_axis_name="c", subcore_axis_name="s")

    @functools.partial(
        pl.kernel, mesh=mesh,
        out_type=jax.ShapeDtypeStruct((b, d, s), jnp.float32),
        compiler_params=pltpu.CompilerParams(use_tc_tiling_on_sc=True,
                                             needs_layout_passes=False),
        scratch_types=[
            pltpu.VMEM((s,), jnp.int32),
            pltpu.VMEM((d, k), jnp.float32),
            pltpu.VMEM((hd, s), jnp.float32),
        ],
    )
    def gk(wt_hbm, idx_hbm, out_hbm, idx_v, wt_v, out_v):
        wid = lax.axis_index("s") * info.num_cores + lax.axis_index("c")
        pltpu.sync_copy(wt_hbm, wt_v)
        pltpu.sync_copy(idx_hbm.at[pl.ds(wid * s, s)], idx_v)
        for h in range(2):
            for dd in range(hd):
                row = jnp.full((nl,), h * hd + dd, jnp.int32)

                @plsc.parallel_loop(0, s // nl, unroll=4)
                def _gather_row(gg, dd=dd, row=row):
                    toks = idx_v[pl.ds(gg * nl, nl)]
                    vals = plsc.load_gather(wt_v, [row, toks])
                    out_v[dd, pl.ds(gg * nl, nl)] = vals
            pltpu.sync_copy(out_v, out_hbm.at[wid, pl.ds(h * hd, hd)])

    return gk(wt, idx)


def kernel(z, W):
    b, s, d = z.shape
    k = W.shape[0]
    zt = jnp.transpose(z, (0, 2, 1))
    wt = jnp.transpose(W, (1, 0))

    idx, loss_acc = pl.pallas_call(
        functools.partial(_vq_tc_body, b, d),
        grid=(b,),
        in_specs=[
            pl.BlockSpec((1, d, s), lambda i: (i, 0, 0)),
            pl.BlockSpec((d, k), lambda i: (0, 0)),
        ],
        out_specs=[
            pl.BlockSpec((s,), lambda i: (i,)),
            pl.BlockSpec((1, 1), lambda i: (0, 0)),
        ],
        out_shape=[
            jax.ShapeDtypeStruct((b * s,), jnp.int32),
            jax.ShapeDtypeStruct((1, 1), jnp.float32),
        ],
    )(zt, wt)

    qt = _sc_gather_t(wt, idx, b, s)

    return (jnp.transpose(qt, (0, 2, 1)), loss_acc[0, 0],
            idx.reshape(b, s))
